# baseline (speedup 1.0000x reference)
import functools

import jax
import jax.numpy as jnp
from jax import lax
from jax.experimental import pallas as pl
from jax.experimental.pallas import tpu as pltpu
from jax.experimental.pallas import tpu_sc as plsc

F32 = jnp.float32
I32 = jnp.int32

T = 2048
D = 768
DFF = 2048
ER = 15
CAP = 171
CP = 176
NR = ER * CP
NW = 32
NPAD = NR + NW
TPW = T // NW
RB = 256
DP = D // 2


BF16 = jnp.bfloat16


U16 = jnp.uint16
U32 = jnp.uint32


def _pack(x):
    lo = lax.bitcast_convert_type(x[:, :DP], U16).astype(U32)
    hi = lax.bitcast_convert_type(x[:, DP:], U16).astype(U32)
    return lax.bitcast_convert_type(lo | (hi << 16), I32)


def _unpack(p):
    u = lax.bitcast_convert_type(p, U32)
    lo = lax.bitcast_convert_type((u & 0xFFFF).astype(U16), BF16)
    hi = lax.bitcast_convert_type((u >> 16).astype(U16), BF16)
    return jnp.concatenate([lo, hi], axis=1)


def _ln_router_body(x_ref, g_ref, b_ref, rw_ref, xn_ref, ds_ref, ci_ref):
    xt = x_ref[...]
    mu = jnp.mean(xt, axis=1, keepdims=True)
    cen = xt - mu
    var = jnp.mean(cen * cen, axis=1, keepdims=True)
    xn = cen * lax.rsqrt(var + 1e-5) * g_ref[...] + b_ref[...]
    xn_ref[...] = _pack(xn.astype(BF16))
    logits = jnp.dot(xn, rw_ref[...], preferred_element_type=F32)
    col = lax.broadcasted_iota(I32, (T, 128), 1)
    lg = jnp.where(col < ER, logits, -1e30)
    m = jnp.max(lg, axis=1, keepdims=True)
    e = jnp.min(jnp.where(lg == m, col, 128), axis=1, keepdims=True)
    oh = (col == e).astype(F32)
    r = lax.broadcasted_iota(I32, (RB, RB), 0)
    c = lax.broadcasted_iota(I32, (RB, RB), 1)
    tri = (r >= c).astype(F32)
    carry = jnp.zeros((1, 128), F32)
    blocks = []
    for i in range(T // RB):
        ohb = oh[i * RB:(i + 1) * RB, :]
        blocks.append(jnp.dot(tri, ohb, preferred_element_type=F32) + carry)
        carry = carry + jnp.sum(ohb, axis=0, keepdims=True)
    pos_incl = jnp.concatenate(blocks, axis=0)
    pos = jnp.sum(oh * pos_incl, axis=1, keepdims=True).astype(I32) - 1
    kept = pos < CAP
    tok = lax.broadcasted_iota(I32, (T, 1), 0)
    ds_ref[...] = jnp.where(kept, e * CP + pos, NR + tok // TPW)
    ci_ref[...] = e * CP + jnp.minimum(pos, CAP)


NF = 4
FC = DFF // NF
NSC = 8
SC_F = DFF // NSC
SRB = T


def _experts_body(d_ref, w1_ref, w3_ref, w2_ref, eo_ref, acc_ref):
    f = pl.program_id(1)
    row = lax.broadcasted_iota(I32, (CP, D), 0)
    xb = jnp.where(row < CAP, _unpack(d_ref[...]), jnp.zeros((), BF16))
    a = jnp.dot(xb, w1_ref[0].astype(BF16), preferred_element_type=F32)
    g = jnp.dot(xb, w3_ref[0].astype(BF16), preferred_element_type=F32)
    h = (a * jax.nn.sigmoid(a) * g).astype(BF16)
    part = jnp.dot(h, w2_ref[0].astype(BF16), preferred_element_type=F32)

    @pl.when(f == 0)
    def _init():
        acc_ref[...] = part

    @pl.when(f > 0)
    def _acc():
        acc_ref[...] += part

    @pl.when(f == NF - 1)
    def _emit():
        eo_ref[...] = _pack(acc_ref[...].astype(BF16))


def _shared_body(x_ref, w1_ref, w3_ref, w2_ref, r_ref, sc_ref, o_ref):
    xb = _unpack(x_ref[...])
    a = jnp.dot(xb, w1_ref[...].astype(BF16), preferred_element_type=F32)
    g = jnp.dot(xb, w3_ref[...].astype(BF16), preferred_element_type=F32)
    h = (a * jax.nn.sigmoid(a) * g).astype(BF16)
    sh = jnp.dot(h, w2_ref[...].astype(BF16), preferred_element_type=F32)
    o_ref[...] = _unpack(r_ref[...]).astype(F32) + sc_ref[0, 0] * sh


def _num_cores():
    return plsc.get_sparse_core_info().num_cores


def _disp_body(xn_hbm, ds_hbm, disp_hbm, idx_v, rows_v, sem):
    wid = lax.axis_index("s") * _num_cores() + lax.axis_index("c")
    base = wid * TPW
    pltpu.sync_copy(ds_hbm.at[pl.ds(base, TPW)], idx_v)
    pltpu.sync_copy(xn_hbm.at[pl.ds(base, TPW)], rows_v)
    pltpu.async_copy(rows_v, disp_hbm.at[idx_v], sem).wait()


def _combine_body(eo_hbm, ci_hbm, routed_hbm, idx_v, rows_v, sem):
    wid = lax.axis_index("s") * _num_cores() + lax.axis_index("c")
    base = wid * TPW
    pltpu.sync_copy(ci_hbm.at[pl.ds(base, TPW)], idx_v)
    pltpu.async_copy(eo_hbm.at[idx_v], rows_v, sem).wait()
    pltpu.sync_copy(rows_v, routed_hbm.at[pl.ds(base, TPW)])


def kernel(x, ln_scale, ln_bias, router_w, sw1, sw3, sw2, ew1, ew3, ew2, shared_scale):
    xt = x.reshape(T, D)
    rw = jnp.pad(router_w, ((0, 0), (0, 128 - ER)))

    xn, ds, ci = pl.pallas_call(
        _ln_router_body,
        out_shape=[
            jax.ShapeDtypeStruct((T, DP), I32),
            jax.ShapeDtypeStruct((T, 1), I32),
            jax.ShapeDtypeStruct((T, 1), I32),
        ],
    )(xt, ln_scale.reshape(1, D), ln_bias.reshape(1, D), rw)
    ds1 = ds.reshape(T)
    ci1 = ci.reshape(T)

    mesh = plsc.VectorSubcoreMesh(core_axis_name="c", subcore_axis_name="s")
    disp = pl.kernel(
        _disp_body,
        mesh=mesh,
        out_type=jax.ShapeDtypeStruct((NPAD, DP), I32),
        scratch_types=[
            pltpu.VMEM((TPW,), I32),
            pltpu.VMEM((TPW, DP), I32),
            pltpu.SemaphoreType.DMA,
        ],
    )(xn, ds1)

    eo = pl.pallas_call(
        _experts_body,
        grid=(ER, NF),
        in_specs=[
            pl.BlockSpec((CP, DP), lambda e, f: (e, 0)),
            pl.BlockSpec((1, D, FC), lambda e, f: (e, 0, f)),
            pl.BlockSpec((1, D, FC), lambda e, f: (e, 0, f)),
            pl.BlockSpec((1, FC, D), lambda e, f: (e, f, 0)),
        ],
        out_specs=pl.BlockSpec((CP, DP), lambda e, f: (e, 0)),
        out_shape=jax.ShapeDtypeStruct((NR, DP), I32),
        scratch_shapes=[pltpu.VMEM((CP, D), F32)],
    )(disp, ew1, ew3, ew2)

    routed = pl.kernel(
        _combine_body,
        mesh=mesh,
        out_type=jax.ShapeDtypeStruct((T, DP), I32),
        scratch_types=[
            pltpu.VMEM((TPW,), I32),
            pltpu.VMEM((TPW, DP), I32),
            pltpu.SemaphoreType.DMA,
        ],
    )(eo, ci1)

    out = pl.pallas_call(
        _shared_body,
        grid=(T // RB,),
        in_specs=[
            pl.BlockSpec((RB, DP), lambda i: (i, 0)),
            pl.BlockSpec((D, DFF), lambda i: (0, 0)),
            pl.BlockSpec((D, DFF), lambda i: (0, 0)),
            pl.BlockSpec((DFF, D), lambda i: (0, 0)),
            pl.BlockSpec((RB, DP), lambda i: (i, 0)),
            pl.BlockSpec(memory_space=pltpu.SMEM),
        ],
        out_specs=pl.BlockSpec((RB, D), lambda i: (i, 0)),
        out_shape=jax.ShapeDtypeStruct((T, D), F32),
    )(xn, sw1, sw3, sw2, routed, shared_scale.reshape(1, 1))

    return out.reshape(1, T, D)


# speedup vs baseline: 1.1024x; 1.1024x over previous
import functools

import jax
import jax.numpy as jnp
from jax import lax
from jax.experimental import pallas as pl
from jax.experimental.pallas import tpu as pltpu
from jax.experimental.pallas import tpu_sc as plsc

F32 = jnp.float32
I32 = jnp.int32

T = 2048
D = 768
DFF = 2048
ER = 15
CAP = 171
CP = 176
NR = ER * CP
NW = 32
NPAD = NR + NW
TPW = T // NW
RB = 256
DP = D // 2


BF16 = jnp.bfloat16


U16 = jnp.uint16
U32 = jnp.uint32


def _pack(x):
    lo = lax.bitcast_convert_type(x[:, :DP], U16).astype(U32)
    hi = lax.bitcast_convert_type(x[:, DP:], U16).astype(U32)
    return lax.bitcast_convert_type(lo | (hi << 16), I32)


def _unpack(p):
    u = lax.bitcast_convert_type(p, U32)
    lo = lax.bitcast_convert_type((u & 0xFFFF).astype(U16), BF16)
    hi = lax.bitcast_convert_type((u >> 16).astype(U16), BF16)
    return jnp.concatenate([lo, hi], axis=1)


def _ln_router_body(x_ref, g_ref, b_ref, rw_ref, xn_ref, ds_ref, ci_ref):
    xt = x_ref[...]
    mu = jnp.mean(xt, axis=1, keepdims=True)
    cen = xt - mu
    var = jnp.mean(cen * cen, axis=1, keepdims=True)
    xn = cen * lax.rsqrt(var + 1e-5) * g_ref[...] + b_ref[...]
    xn_ref[...] = _pack(xn.astype(BF16))
    logits = jnp.dot(xn, rw_ref[...], preferred_element_type=F32)
    col = lax.broadcasted_iota(I32, (T, 128), 1)
    lg = jnp.where(col < ER, logits, -1e30)
    m = jnp.max(lg, axis=1, keepdims=True)
    e = jnp.min(jnp.where(lg == m, col, 128), axis=1, keepdims=True)
    oh = (col == e).astype(F32)
    r = lax.broadcasted_iota(I32, (RB, RB), 0)
    c = lax.broadcasted_iota(I32, (RB, RB), 1)
    tri = (r >= c).astype(F32)
    carry = jnp.zeros((1, 128), F32)
    blocks = []
    for i in range(T // RB):
        ohb = oh[i * RB:(i + 1) * RB, :]
        blocks.append(jnp.dot(tri, ohb, preferred_element_type=F32) + carry)
        carry = carry + jnp.sum(ohb, axis=0, keepdims=True)
    pos_incl = jnp.concatenate(blocks, axis=0)
    pos = jnp.sum(oh * pos_incl, axis=1, keepdims=True).astype(I32) - 1
    kept = pos < CAP
    tok = lax.broadcasted_iota(I32, (T, 1), 0)
    ds_ref[...] = jnp.where(kept, e * CP + pos, NR + tok // TPW)
    ci_ref[...] = e * CP + jnp.minimum(pos, CAP)


NF = 2
FC = DFF // NF
NSC = 8
SC_F = DFF // NSC
SRB = T


def _experts_body(d_ref, w1_ref, w3_ref, w2_ref, eo_ref, acc_ref):
    f = pl.program_id(1)
    row = lax.broadcasted_iota(I32, (CP, D), 0)
    xb = jnp.where(row < CAP, _unpack(d_ref[...]), jnp.zeros((), BF16))
    a = jnp.dot(xb, w1_ref[0].astype(BF16), preferred_element_type=F32)
    g = jnp.dot(xb, w3_ref[0].astype(BF16), preferred_element_type=F32)
    h = (a * jax.nn.sigmoid(a) * g).astype(BF16)
    part = jnp.dot(h, w2_ref[0].astype(BF16), preferred_element_type=F32)

    @pl.when(f == 0)
    def _init():
        acc_ref[...] = part

    @pl.when(f > 0)
    def _acc():
        acc_ref[...] += part

    @pl.when(f == NF - 1)
    def _emit():
        eo_ref[...] = _pack(acc_ref[...].astype(BF16))


SHB = 512


def _shared_body(x_ref, w1_ref, w3_ref, w2_ref, r_ref, sc_ref, o_ref):
    xb = _unpack(x_ref[...])
    a = jnp.dot(xb, w1_ref[...].astype(BF16), preferred_element_type=F32)
    g = jnp.dot(xb, w3_ref[...].astype(BF16), preferred_element_type=F32)
    h = (a * jax.nn.sigmoid(a) * g).astype(BF16)
    sh = jnp.dot(h, w2_ref[...].astype(BF16), preferred_element_type=F32)
    o_ref[...] = _unpack(r_ref[...]).astype(F32) + sc_ref[0, 0] * sh


def _num_cores():
    return plsc.get_sparse_core_info().num_cores


def _disp_body(xn_hbm, ds_hbm, disp_hbm, idx_v, rows_v, sem):
    wid = lax.axis_index("s") * _num_cores() + lax.axis_index("c")
    base = wid * TPW
    pltpu.sync_copy(ds_hbm.at[pl.ds(base, TPW)], idx_v)
    pltpu.sync_copy(xn_hbm.at[pl.ds(base, TPW)], rows_v)
    pltpu.async_copy(rows_v, disp_hbm.at[idx_v], sem).wait()


def _combine_body(eo_hbm, ci_hbm, routed_hbm, idx_v, rows_v, sem):
    wid = lax.axis_index("s") * _num_cores() + lax.axis_index("c")
    base = wid * TPW
    pltpu.sync_copy(ci_hbm.at[pl.ds(base, TPW)], idx_v)
    pltpu.async_copy(eo_hbm.at[idx_v], rows_v, sem).wait()
    pltpu.sync_copy(rows_v, routed_hbm.at[pl.ds(base, TPW)])


def kernel(x, ln_scale, ln_bias, router_w, sw1, sw3, sw2, ew1, ew3, ew2, shared_scale):
    xt = x.reshape(T, D)
    rw = jnp.pad(router_w, ((0, 0), (0, 128 - ER)))

    xn, ds, ci = pl.pallas_call(
        _ln_router_body,
        out_shape=[
            jax.ShapeDtypeStruct((T, DP), I32),
            jax.ShapeDtypeStruct((T, 1), I32),
            jax.ShapeDtypeStruct((T, 1), I32),
        ],
    )(xt, ln_scale.reshape(1, D), ln_bias.reshape(1, D), rw)
    ds1 = ds.reshape(T)
    ci1 = ci.reshape(T)

    mesh = plsc.VectorSubcoreMesh(core_axis_name="c", subcore_axis_name="s")
    disp = pl.kernel(
        _disp_body,
        mesh=mesh,
        out_type=jax.ShapeDtypeStruct((NPAD, DP), I32),
        scratch_types=[
            pltpu.VMEM((TPW,), I32),
            pltpu.VMEM((TPW, DP), I32),
            pltpu.SemaphoreType.DMA,
        ],
    )(xn, ds1)

    eo = pl.pallas_call(
        _experts_body,
        grid=(ER, NF),
        in_specs=[
            pl.BlockSpec((CP, DP), lambda e, f: (e, 0)),
            pl.BlockSpec((1, D, FC), lambda e, f: (e, 0, f)),
            pl.BlockSpec((1, D, FC), lambda e, f: (e, 0, f)),
            pl.BlockSpec((1, FC, D), lambda e, f: (e, f, 0)),
        ],
        out_specs=pl.BlockSpec((CP, DP), lambda e, f: (e, 0)),
        out_shape=jax.ShapeDtypeStruct((NR, DP), I32),
        scratch_shapes=[pltpu.VMEM((CP, D), F32)],
    )(disp, ew1, ew3, ew2)

    routed = pl.kernel(
        _combine_body,
        mesh=mesh,
        out_type=jax.ShapeDtypeStruct((T, DP), I32),
        scratch_types=[
            pltpu.VMEM((TPW,), I32),
            pltpu.VMEM((TPW, DP), I32),
            pltpu.SemaphoreType.DMA,
        ],
    )(eo, ci1)

    out = pl.pallas_call(
        _shared_body,
        grid=(T // SHB,),
        in_specs=[
            pl.BlockSpec((SHB, DP), lambda i: (i, 0)),
            pl.BlockSpec((D, DFF), lambda i: (0, 0)),
            pl.BlockSpec((D, DFF), lambda i: (0, 0)),
            pl.BlockSpec((DFF, D), lambda i: (0, 0)),
            pl.BlockSpec((SHB, DP), lambda i: (i, 0)),
            pl.BlockSpec(memory_space=pltpu.SMEM),
        ],
        out_specs=pl.BlockSpec((SHB, D), lambda i: (i, 0)),
        out_shape=jax.ShapeDtypeStruct((T, D), F32),
    )(xn, sw1, sw3, sw2, routed, shared_scale.reshape(1, 1))

    return out.reshape(1, T, D)
